# baseline (speedup 1.0000x reference)
import jax
import jax.numpy as jnp
from jax import lax
from jax.experimental import pallas as pl
from jax.experimental.pallas import tpu as pltpu
from jax.experimental.pallas import tpu_sc as plsc

WEIGHT_DECAY_ = 0.01

NC = 2
NS = 16
L = 16
NW = NC * NS
B = 16384
D = 64
PER_W = B // NW
CHUNK = 64
NCHUNK = PER_W // CHUNK
NB = 2


def _sc_body(u_hbm, i_hbm, j_hbm, wp_hbm, hp_hbm, p_hbm, sq_hbm,
             ui_v, ii_v, ji_v, ru_v, ri_v, rj_v,
             ue_v, ie_v, je_v, pb_v, sq_v, sem):
    wid = lax.axis_index("s") * NC + lax.axis_index("c")
    base = wid * PER_W

    for src, dst, rdst in ((u_hbm, ui_v, ru_v), (i_hbm, ii_v, ri_v),
                           (j_hbm, ji_v, rj_v)):
        for c in range(NCHUNK):
            pltpu.sync_copy(src.at[pl.ds(base + c * CHUNK, CHUNK)],
                            dst.at[c, pl.ds(0, CHUNK)])
            for q in range(CHUNK // L):
                sl = pl.ds(q * L, L)
                rdst[c, sl] = lax.shift_right_logical(dst[c, sl], 1)

    def fire(c):
        b = c % NB
        return [
            pltpu.async_copy(wp_hbm.at[ru_v.at[c]], ue_v.at[b], sem),
            pltpu.async_copy(hp_hbm.at[ri_v.at[c]], ie_v.at[b], sem),
            pltpu.async_copy(hp_hbm.at[rj_v.at[c]], je_v.at[b], sem),
        ]

    inflight = fire(0)
    sq_acc = jnp.zeros((L,), jnp.float32)

    for c in range(NCHUNK):
        for cp in inflight:
            cp.wait()
        if c + 1 < NCHUNK:
            inflight = fire(c + 1)
        b = c % NB

        def bodyc(g, sq_acc, c=c, b=b):
            s0 = g * L
            zero = jnp.zeros((L,), jnp.float32)
            uf = (ui_v[c, pl.ds(s0, L)] & 1).astype(jnp.float32)
            vf = (ii_v[c, pl.ds(s0, L)] & 1).astype(jnp.float32)
            wf = (ji_v[c, pl.ds(s0, L)] & 1).astype(jnp.float32)
            for k in range(L):
                s = s0 + k
                pu = zero + uf[k]
                pi = zero + vf[k]
                pj = zero + wf[k]
                p = zero
                for l in range(D // L):
                    lo = pl.ds(l * L, L)
                    hi = pl.ds(D + l * L, L)
                    a0 = ue_v[b, s, lo]
                    ue = a0 + (ue_v[b, s, hi] - a0) * pu
                    b0 = ie_v[b, s, lo]
                    ie = b0 + (ie_v[b, s, hi] - b0) * pi
                    c0 = je_v[b, s, lo]
                    je = c0 + (je_v[b, s, hi] - c0) * pj
                    p = p + ue * (ie - je)
                    sq_acc = sq_acc + (ue * ue + ie * ie + je * je)
                pb_v[c * CHUNK + s, pl.ds(0, L)] = p
            return sq_acc

        sq_acc = lax.fori_loop(0, CHUNK // L, bodyc, sq_acc)

    sq_v[...] = sq_acc
    pltpu.sync_copy(pb_v, p_hbm.at[wid])
    pltpu.sync_copy(sq_v, sq_hbm.at[wid])


def _tc_body(p_ref, s_ref, sq_ref, o_ref):
    x = lax.dot_general(p_ref[...], s_ref[...], (((1,), (0,)), ((), ())),
                        precision=lax.Precision.HIGHEST,
                        preferred_element_type=jnp.float32)
    ls = jnp.where(x >= 0.0,
                   -jnp.log1p(jnp.exp(-x)),
                   x - jnp.log1p(jnp.exp(x)))
    col = lax.broadcasted_iota(jnp.int32, ls.shape, 1)
    ls = jnp.where(col < 8, ls, 0.0)
    o_ref[0, 0] = WEIGHT_DECAY_ * jnp.sum(sq_ref[...]) - jnp.sum(ls)


@jax.jit
def kernel(u, i, j, W, H):
    u1 = u.astype(jnp.int32)
    i1 = i.astype(jnp.int32)
    j1 = j.astype(jnp.int32)
    Wp = W.reshape(50000, 128)
    Hp = H.reshape(50000, 128)

    mesh = plsc.VectorSubcoreMesh(core_axis_name="c", subcore_axis_name="s",
                                  num_cores=NC, num_subcores=NS)
    sc = pl.kernel(
        _sc_body,
        out_type=(jax.ShapeDtypeStruct((NW, PER_W, L), jnp.float32),
                  jax.ShapeDtypeStruct((NW, L), jnp.float32)),
        mesh=mesh,
        scratch_types=[
            pltpu.VMEM((NCHUNK, CHUNK + L), jnp.int32),
            pltpu.VMEM((NCHUNK, CHUNK + L), jnp.int32),
            pltpu.VMEM((NCHUNK, CHUNK + L), jnp.int32),
            pltpu.VMEM((NCHUNK, CHUNK), jnp.int32),
            pltpu.VMEM((NCHUNK, CHUNK), jnp.int32),
            pltpu.VMEM((NCHUNK, CHUNK), jnp.int32),
            pltpu.VMEM((NB, CHUNK, 2 * D), jnp.float32),
            pltpu.VMEM((NB, CHUNK, 2 * D), jnp.float32),
            pltpu.VMEM((NB, CHUNK, 2 * D), jnp.float32),
            pltpu.VMEM((PER_W, L), jnp.float32),
            pltpu.VMEM((L,), jnp.float32),
            pltpu.SemaphoreType.DMA,
        ],
    )
    p, sq = sc(u1, i1, j1, Wp, Hp)

    sel = (jnp.arange(128)[:, None] // L ==
           jnp.arange(128)[None, :]).astype(jnp.float32)

    loss = pl.pallas_call(
        _tc_body,
        out_shape=jax.ShapeDtypeStruct((1, 1), jnp.float32),
        out_specs=pl.BlockSpec(memory_space=pltpu.SMEM),
    )(p.reshape(B // 8, 8 * L), sel, sq.reshape(NW * L // 128, 128))
    return loss[0, 0]


# speedup vs baseline: 1.1955x; 1.1955x over previous
import jax
import jax.numpy as jnp
from jax import lax
from jax.experimental import pallas as pl
from jax.experimental.pallas import tpu as pltpu
from jax.experimental.pallas import tpu_sc as plsc

WEIGHT_DECAY_ = 0.01

NC = 2
NS = 16
L = 16
NW = NC * NS
B = 16384
D = 64
PER_W = B // NW
CHUNK = 128
NCHUNK = PER_W // CHUNK
NB = 2


def _sc_body(u_hbm, i_hbm, j_hbm, w_hbm, h_hbm, p_hbm, sq_hbm,
             ui_v, ii_v, ji_v, ue_v, ie_v, je_v, pb_v, sq_v, sem):
    wid = lax.axis_index("s") * NC + lax.axis_index("c")
    base = wid * PER_W

    for src, dst in ((u_hbm, ui_v), (i_hbm, ii_v), (j_hbm, ji_v)):
        for c in range(NCHUNK):
            pltpu.sync_copy(src.at[pl.ds(base + c * CHUNK, CHUNK)], dst.at[c])

    def fire(c):
        b = c % NB
        return [
            pltpu.async_copy(w_hbm.at[ui_v.at[c]], ue_v.at[b], sem),
            pltpu.async_copy(h_hbm.at[ii_v.at[c]], ie_v.at[b], sem),
            pltpu.async_copy(h_hbm.at[ji_v.at[c]], je_v.at[b], sem),
        ]

    inflight = fire(0)
    sq_acc = jnp.zeros((L,), jnp.float32)

    for c in range(NCHUNK):
        for cp in inflight:
            cp.wait()
        if c + 1 < NCHUNK:
            inflight = fire(c + 1)
        b = c % NB

        def bodyc(h, sq_acc, c=c, b=b):
            for k in range(4):
                s = h * 4 + k
                p = jnp.zeros((L,), jnp.float32)
                for l in range(D // L):
                    sl = pl.ds(l * L, L)
                    ue = ue_v[b, s, sl]
                    ie = ie_v[b, s, sl]
                    je = je_v[b, s, sl]
                    p = p + ue * (ie - je)
                    sq_acc = sq_acc + (ue * ue + ie * ie + je * je)
                pb_v[c * CHUNK + s, pl.ds(0, L)] = p
            return sq_acc

        sq_acc = lax.fori_loop(0, CHUNK // 4, bodyc, sq_acc)

    sq_v[...] = sq_acc
    pltpu.sync_copy(pb_v, p_hbm.at[wid])
    pltpu.sync_copy(sq_v, sq_hbm.at[wid])


def _tc_body(p_ref, s_ref, sq_ref, o_ref):
    x = lax.dot_general(p_ref[...], s_ref[...], (((1,), (0,)), ((), ())),
                        precision=lax.Precision.HIGHEST,
                        preferred_element_type=jnp.float32)
    ls = jnp.where(x >= 0.0,
                   -jnp.log1p(jnp.exp(-x)),
                   x - jnp.log1p(jnp.exp(x)))
    col = lax.broadcasted_iota(jnp.int32, ls.shape, 1)
    ls = jnp.where(col < 8, ls, 0.0)
    o_ref[0, 0] = WEIGHT_DECAY_ * jnp.sum(sq_ref[...]) - jnp.sum(ls)


@jax.jit
def kernel(u, i, j, W, H):
    u1 = u.astype(jnp.int32)
    i1 = i.astype(jnp.int32)
    j1 = j.astype(jnp.int32)
    Wf, Hf = jax.lax.optimization_barrier((W.reshape(-1), H.reshape(-1)))
    Wv = Wf.reshape(100000, D)
    Hv = Hf.reshape(100000, D)

    mesh = plsc.VectorSubcoreMesh(core_axis_name="c", subcore_axis_name="s",
                                  num_cores=NC, num_subcores=NS)
    sc = pl.kernel(
        _sc_body,
        out_type=(jax.ShapeDtypeStruct((NW, PER_W, L), jnp.float32),
                  jax.ShapeDtypeStruct((NW, L), jnp.float32)),
        mesh=mesh,
        compiler_params=pltpu.CompilerParams(use_tc_tiling_on_sc=False),
        scratch_types=[
            pltpu.VMEM((NCHUNK, CHUNK), jnp.int32),
            pltpu.VMEM((NCHUNK, CHUNK), jnp.int32),
            pltpu.VMEM((NCHUNK, CHUNK), jnp.int32),
            pltpu.VMEM((NB, CHUNK, D), jnp.float32),
            pltpu.VMEM((NB, CHUNK, D), jnp.float32),
            pltpu.VMEM((NB, CHUNK, D), jnp.float32),
            pltpu.VMEM((PER_W, L), jnp.float32),
            pltpu.VMEM((L,), jnp.float32),
            pltpu.SemaphoreType.DMA,
        ],
    )
    p, sq = sc(u1, i1, j1, Wv, Hv)

    sel = (jnp.arange(128)[:, None] // L ==
           jnp.arange(128)[None, :]).astype(jnp.float32)

    loss = pl.pallas_call(
        _tc_body,
        out_shape=jax.ShapeDtypeStruct((1, 1), jnp.float32),
        out_specs=pl.BlockSpec(memory_space=pltpu.SMEM),
    )(p.reshape(B // 8, 8 * L), sel, sq.reshape(NW * L // 128, 128))
    return loss[0, 0]


# speedup vs baseline: 1.2289x; 1.0279x over previous
import jax
import jax.numpy as jnp
from jax import lax
from jax.experimental import pallas as pl
from jax.experimental.pallas import tpu as pltpu
from jax.experimental.pallas import tpu_sc as plsc

WEIGHT_DECAY_ = 0.01

NC = 2
NS = 16
L = 16
NW = NC * NS
B = 16384
D = 64
PER_W = B // NW
CHUNK = 128
NCHUNK = PER_W // CHUNK
NB = 2


def _sc_body(u_hbm, i_hbm, j_hbm, w_hbm, h_hbm, p_hbm, sq_hbm,
             ui_v, ii_v, ji_v, ue_v, ie_v, je_v, pb_v, sq_v, sem):
    wid = lax.axis_index("s") * NC + lax.axis_index("c")
    base = wid * PER_W

    staged = [
        pltpu.async_copy(src.at[pl.ds(base + c * CHUNK, CHUNK)], dst.at[c],
                         sem)
        for src, dst in ((u_hbm, ui_v), (i_hbm, ii_v), (j_hbm, ji_v))
        for c in range(NCHUNK)
    ]
    for cp in staged:
        cp.wait()

    def fire(c):
        b = c % NB
        return [
            pltpu.async_copy(w_hbm.at[ui_v.at[c]], ue_v.at[b], sem),
            pltpu.async_copy(h_hbm.at[ii_v.at[c]], ie_v.at[b], sem),
            pltpu.async_copy(h_hbm.at[ji_v.at[c]], je_v.at[b], sem),
        ]

    inflight = fire(0)
    sq_acc = jnp.zeros((L,), jnp.float32)

    for c in range(NCHUNK):
        for cp in inflight:
            cp.wait()
        if c + 1 < NCHUNK:
            inflight = fire(c + 1)
        b = c % NB

        def bodyc(h, sq_acc, c=c, b=b):
            for k in range(4):
                s = h * 4 + k
                p = jnp.zeros((L,), jnp.float32)
                for l in range(D // L):
                    sl = pl.ds(l * L, L)
                    ue = ue_v[b, s, sl]
                    ie = ie_v[b, s, sl]
                    je = je_v[b, s, sl]
                    p = p + ue * (ie - je)
                    sq_acc = sq_acc + (ue * ue + ie * ie + je * je)
                pb_v[c * CHUNK + s, pl.ds(0, L)] = p
            return sq_acc

        sq_acc = lax.fori_loop(0, CHUNK // 4, bodyc, sq_acc)

    sq_v[...] = sq_acc
    pltpu.sync_copy(pb_v, p_hbm.at[wid])
    pltpu.sync_copy(sq_v, sq_hbm.at[wid])


def _tc_body(p_ref, s_ref, sq_ref, o_ref):
    x = lax.dot_general(p_ref[...], s_ref[...], (((1,), (0,)), ((), ())),
                        precision=lax.Precision.HIGHEST,
                        preferred_element_type=jnp.float32)
    ls = jnp.where(x >= 0.0,
                   -jnp.log1p(jnp.exp(-x)),
                   x - jnp.log1p(jnp.exp(x)))
    col = lax.broadcasted_iota(jnp.int32, ls.shape, 1)
    ls = jnp.where(col < 8, ls, 0.0)
    o_ref[0, 0] = WEIGHT_DECAY_ * jnp.sum(sq_ref[...]) - jnp.sum(ls)


@jax.jit
def kernel(u, i, j, W, H):
    u1 = u.astype(jnp.int32)
    i1 = i.astype(jnp.int32)
    j1 = j.astype(jnp.int32)
    Wf, Hf = jax.lax.optimization_barrier((W.reshape(-1), H.reshape(-1)))
    Wv = Wf.reshape(100000, D)
    Hv = Hf.reshape(100000, D)

    mesh = plsc.VectorSubcoreMesh(core_axis_name="c", subcore_axis_name="s",
                                  num_cores=NC, num_subcores=NS)
    sc = pl.kernel(
        _sc_body,
        out_type=(jax.ShapeDtypeStruct((NW, PER_W, L), jnp.float32),
                  jax.ShapeDtypeStruct((NW, L), jnp.float32)),
        mesh=mesh,
        compiler_params=pltpu.CompilerParams(use_tc_tiling_on_sc=False),
        scratch_types=[
            pltpu.VMEM((NCHUNK, CHUNK), jnp.int32),
            pltpu.VMEM((NCHUNK, CHUNK), jnp.int32),
            pltpu.VMEM((NCHUNK, CHUNK), jnp.int32),
            pltpu.VMEM((NB, CHUNK, D), jnp.float32),
            pltpu.VMEM((NB, CHUNK, D), jnp.float32),
            pltpu.VMEM((NB, CHUNK, D), jnp.float32),
            pltpu.VMEM((PER_W, L), jnp.float32),
            pltpu.VMEM((L,), jnp.float32),
            pltpu.SemaphoreType.DMA,
        ],
    )
    p, sq = sc(u1, i1, j1, Wv, Hv)

    sel = (jnp.arange(128)[:, None] // L ==
           jnp.arange(128)[None, :]).astype(jnp.float32)

    loss = pl.pallas_call(
        _tc_body,
        out_shape=jax.ShapeDtypeStruct((1, 1), jnp.float32),
        out_specs=pl.BlockSpec(memory_space=pltpu.SMEM),
    )(p.reshape(B // 8, 8 * L), sel, sq.reshape(NW * L // 128, 128))
    return loss[0, 0]
